# baseline (speedup 1.0000x reference)
import jax
import jax.numpy as jnp
from jax import lax
from jax.experimental import pallas as pl
from jax.experimental.pallas import tpu as pltpu
from jax.experimental.pallas import tpu_sc as plsc

_N, _E, _T, _D = 19, 342, 1025, 512
_EP = 512
_NSEG = 32
_NEG = -1e30


def _leaky(z):
    return jnp.where(z > 0, z, 0.2 * z)


_GDN = lax.GatherDimensionNumbers(offset_dims=(), collapsed_slice_dims=(0,),
                                  start_index_map=(0,))


def _perm(v, idx):
    return lax.gather(v, idx[:, None], _GDN, slice_sizes=(1,),
                      mode=lax.GatherScatterMode.PROMISE_IN_BOUNDS)


def _bfly_sum(v, lane):
    for st in (8, 4, 2, 1):
        v = v + _perm(v, lane ^ st)
    return v


def _bfly_max(v, lane):
    for st in (8, 4, 2, 1):
        v = jnp.maximum(v, _perm(v, lane ^ st))
    return v


def _proj_body(ei_ref, x_ref, wl_ref, wr_ref,
               xl_ref, xr_ref, sidx_ref, didx_ref, dseg_ref):
    x = x_ref[:]
    xl_ref[:] = jnp.dot(x, wl_ref[:], preferred_element_type=jnp.float32)
    xr_ref[:] = jnp.dot(x, wr_ref[:], preferred_element_type=jnp.float32)
    src_row = ei_ref[0:1, :]
    dst_row = ei_ref[1:2, :]
    zpad = jnp.zeros((1, _EP - _E), jnp.int32)
    sidx_ref[:] = jnp.concatenate([src_row, zpad], axis=1)
    didx_ref[:] = jnp.concatenate([dst_row, zpad], axis=1)
    dseg_ref[:] = jnp.concatenate([dst_row, zpad + (_NSEG - 1)], axis=1)


def _sc_alpha(xl_hbm, xr_hbm, sidx_hbm, didx_hbm, dseg_hbm, a_hbm,
              alpha_hbm,
              av, sv, dv, dall, eall, ebuf, abuf, tmp,
              xls, xrd, aall, e_sh, astage_sh, sem1, sem2):
    cid = lax.axis_index("c")
    sid = lax.axis_index("s")
    lane = lax.iota(jnp.int32, 16)

    @pl.when(cid == 0)
    def _p1():
        pltpu.sync_copy(a_hbm, av)
        pltpu.sync_copy(dseg_hbm, dall)
        for b in range(2):
            base = (sid * 2 + b) * 16
            pltpu.sync_copy(sidx_hbm.at[pl.ds(base, 16)], sv)
            pltpu.sync_copy(didx_hbm.at[pl.ds(base, 16)], dv)
            pltpu.async_copy(xl_hbm.at[sv], xls, sem1).wait()
            pltpu.async_copy(xr_hbm.at[dv], xrd, sem2).wait()
            e_vec = jnp.zeros((16,), jnp.float32)
            for i in range(16):
                def kbody(k, acc, _i=i):
                    sl = pl.ds(k * 16, 16)
                    return acc + _leaky(xls[_i, sl] + xrd[_i, sl]) * av[sl]
                acc = lax.fori_loop(0, _D // 16, kbody,
                                    jnp.zeros((16,), jnp.float32))
                e_vec = jnp.where(lane == i, _bfly_sum(acc, lane), e_vec)
            ebuf[...] = e_vec
            pltpu.sync_copy(ebuf, e_sh.at[pl.ds(base, 16)])

    plsc.subcore_barrier()

    @pl.when(cid == 0)
    def _p2():
        pltpu.sync_copy(e_sh, eall)
        for c in range(_EP // 16):
            aall[pl.ds(c * 16, 16)] = jnp.zeros((16,), jnp.float32)
        for t in range(2):
            seg = sid + 16 * t
            segspl = jnp.full((16,), seg, jnp.int32)
            mvec = jnp.full((16,), _NEG, jnp.float32)
            for c in range(_EP // 16):
                sl = pl.ds(c * 16, 16)
                mvec = jnp.maximum(mvec,
                                   jnp.where(dall[sl] == segspl, eall[sl],
                                             _NEG))
            mspl = _bfly_max(mvec, lane)
            svec = jnp.zeros((16,), jnp.float32)
            for c in range(_EP // 16):
                sl = pl.ds(c * 16, 16)
                svec = svec + jnp.where(dall[sl] == segspl,
                                        jnp.exp(eall[sl] - mspl), 0.0)
            sspl = _bfly_sum(svec, lane)
            for c in range(_EP // 16):
                sl = pl.ds(c * 16, 16)
                contrib = jnp.where(dall[sl] == segspl,
                                    jnp.exp(eall[sl] - mspl) / (sspl + 1e-16),
                                    0.0)
                aall[sl] = aall[sl] + contrib
        pltpu.sync_copy(aall, astage_sh.at[sid])

    plsc.subcore_barrier()

    @pl.when(cid == 0)
    def _p3():
        for b in range(2):
            base = (sid * 2 + b) * 16
            alpha_vec = jnp.zeros((16,), jnp.float32)
            for w in range(16):
                pltpu.sync_copy(astage_sh.at[w, pl.ds(base, 16)], tmp)
                alpha_vec = alpha_vec + tmp[...]
            abuf[...] = alpha_vec
            pltpu.sync_copy(abuf, alpha_hbm.at[pl.ds(base, 16)])


def _tail_body(ei_ref, alpha_ref, xl1_ref, b1_ref, wl2_ref, wr2_ref,
               a2_ref, b2_ref, mmse_ref, wm_ref, bm_ref, dec_ref):
    src_row = ei_ref[0:1, :]
    dst_row = ei_ref[1:2, :]
    ion = lax.broadcasted_iota(jnp.int32, (_N, _E), 0)
    srcf = (ion == src_row).astype(jnp.float32)
    dmask = ion == dst_row
    dstf = dmask.astype(jnp.float32)
    alpha1 = alpha_ref[0:1, 0:_E]
    aw1 = jnp.where(dmask, jnp.broadcast_to(alpha1, (_N, _E)), 0.0)
    adj1 = lax.dot_general(aw1, srcf, (((1,), (1,)), ((), ())),
                           preferred_element_type=jnp.float32)
    h1 = jnp.dot(adj1, xl1_ref[:], preferred_element_type=jnp.float32) \
        + b1_ref[:].reshape(1, _D)
    xl = jnp.dot(h1, wl2_ref[:], preferred_element_type=jnp.float32)
    xr = jnp.dot(h1, wr2_ref[:], preferred_element_type=jnp.float32)
    xls = lax.dot_general(srcf, xl, (((0,), (0,)), ((), ())),
                          preferred_element_type=jnp.float32)
    xrd = lax.dot_general(dstf, xr, (((0,), (0,)), ((), ())),
                          preferred_element_type=jnp.float32)
    he = _leaky(xls + xrd)
    e_row = lax.dot_general(a2_ref[:].reshape(1, _D), he,
                            (((1,), (1,)), ((), ())),
                            preferred_element_type=jnp.float32)
    eb = jnp.broadcast_to(e_row, (_N, _E))
    m = jnp.max(jnp.where(dmask, eb, _NEG), axis=1, keepdims=True)
    mdst = jnp.max(jnp.where(dmask, jnp.broadcast_to(m, (_N, _E)), _NEG),
                   axis=0, keepdims=True)
    ex = jnp.exp(jnp.minimum(e_row - mdst, 0.0))
    s = jnp.sum(jnp.where(dmask, jnp.broadcast_to(ex, (_N, _E)), 0.0),
                axis=1, keepdims=True)
    sdst = jnp.sum(jnp.where(dmask, jnp.broadcast_to(s, (_N, _E)), 0.0),
                   axis=0, keepdims=True)
    alpha = ex / (sdst + 1e-16)
    aw = jnp.where(dmask, jnp.broadcast_to(alpha, (_N, _E)), 0.0)
    adj = lax.dot_general(aw, srcf, (((1,), (1,)), ((), ())),
                          preferred_element_type=jnp.float32)
    h2 = jnp.dot(adj, xl, preferred_element_type=jnp.float32) \
        + b2_ref[:].reshape(1, _D)
    gf = h2 + (mmse_ref[0] * wm_ref[:] + bm_ref[:].reshape(1, _D))
    dec = lax.dot_general(gf, gf, (((1,), (1,)), ((), ())),
                          preferred_element_type=jnp.float32)
    dec_ref[:] = jax.nn.sigmoid(dec)


def kernel(x, edge_index, mmse, Wl1, Wr1, a1, b1, Wl2, Wr2, a2, b2, Wm, bm,
           W11, b11, W12, b12, W21, b21, W22, b22, Wp, bp):
    xl, xr, sidx, didx, dseg = pl.pallas_call(
        _proj_body,
        out_shape=[jax.ShapeDtypeStruct((_N, _D), jnp.float32),
                   jax.ShapeDtypeStruct((_N, _D), jnp.float32),
                   jax.ShapeDtypeStruct((1, _EP), jnp.int32),
                   jax.ShapeDtypeStruct((1, _EP), jnp.int32),
                   jax.ShapeDtypeStruct((1, _EP), jnp.int32)],
    )(edge_index, x, Wl1, Wr1)

    mesh = plsc.VectorSubcoreMesh(core_axis_name="c", subcore_axis_name="s")
    sc_fn = pl.kernel(
        _sc_alpha,
        out_type=[jax.ShapeDtypeStruct((_EP,), jnp.float32)],
        mesh=mesh,
        scratch_types=[
            pltpu.VMEM((_D,), jnp.float32),
            pltpu.VMEM((16,), jnp.int32),
            pltpu.VMEM((16,), jnp.int32),
            pltpu.VMEM((_EP,), jnp.int32),
            pltpu.VMEM((_EP,), jnp.float32),
            pltpu.VMEM((16,), jnp.float32),
            pltpu.VMEM((16,), jnp.float32),
            pltpu.VMEM((16,), jnp.float32),
            pltpu.VMEM((16, _D), jnp.float32),
            pltpu.VMEM((16, _D), jnp.float32),
            pltpu.VMEM((_EP,), jnp.float32),
            pltpu.VMEM_SHARED((_EP,), jnp.float32),
            pltpu.VMEM_SHARED((16, _EP), jnp.float32),
            pltpu.SemaphoreType.DMA,
            pltpu.SemaphoreType.DMA,
        ],
    )
    alpha_sc = sc_fn(xl, xr, sidx.reshape(_EP), didx.reshape(_EP),
                     dseg.reshape(_EP), a1)
    if isinstance(alpha_sc, (list, tuple)):
        alpha_sc = alpha_sc[0]

    dec = pl.pallas_call(
        _tail_body,
        out_shape=jax.ShapeDtypeStruct((_N, _N), jnp.float32),
    )(edge_index, alpha_sc.reshape(1, _EP), xl, b1, Wl2, Wr2, a2, b2,
      mmse, Wm, bm)
    return dec, alpha_sc[:_E]
